# baseline (speedup 1.0000x reference)
import functools

import jax
import jax.numpy as jnp
from jax import lax
from jax.experimental import pallas as pl
from jax.experimental.pallas import tpu as pltpu
from jax.experimental.pallas import tpu_sc as plsc

N, T, E = 64, 512, 768
D, G = 8, 8
DG = D * G
RT = 2
TQ = T // RT
RE = RT * E
NC, NS = 2, 16
NW = NC * NS
TPW = T // NW
OPW = TPW // RT
NGATH = OPW * D
L = 16
CPR = RE // L


def _pooler_body(p_hbm, idx_hbm, out_hbm, base_v, idx_v, rows_v, out_v,
                 gsem, osem):
    cid = lax.axis_index("c")
    sid = lax.axis_index("s")
    wid = sid * NC + cid
    t0 = wid * TPW
    oct0 = wid * OPW

    pltpu.sync_copy(idx_hbm, base_v)
    for o in range(OPW):
        for k in range(DG // L):
            sl = pl.ds(k * L, L)
            idx_v[o, sl] = base_v[sl] + (oct0 + o)

    def gather_start(i, b):
        o = i // D
        d = i - o * D
        off = d * G
        if not isinstance(off, int):
            off = pl.multiple_of(off, 8)
        pltpu.async_copy(
            p_hbm.at[idx_v.at[o, pl.ds(off, G)]],
            rows_v.at[b], gsem)

    def gather_wait(b):
        pltpu.make_async_copy(
            p_hbm.at[idx_v.at[0, pl.ds(0, G)]], rows_v.at[b], gsem).wait()

    gather_start(0, 0)
    gather_start(1, 1)

    def i2_body(io, carry):
        for b in range(2):
            i = io * 2 + b
            o = i // D
            d = i - o * D
            gather_wait(b)
            @pl.when(i >= 2)
            def _():
                pltpu.make_async_copy(
                    out_v.at[b], out_hbm.at[pl.ds(t0, RT), 0], osem).wait()

            def rt_body(rt, carry2):
                r0 = rt * E
                for c in range(E // L):
                    sl = pl.ds(r0 + c * L, L)
                    m = jnp.maximum(
                        jnp.maximum(
                            jnp.maximum(rows_v[b, 0, sl], rows_v[b, 1, sl]),
                            jnp.maximum(rows_v[b, 2, sl], rows_v[b, 3, sl]),
                        ),
                        jnp.maximum(
                            jnp.maximum(rows_v[b, 4, sl], rows_v[b, 5, sl]),
                            jnp.maximum(rows_v[b, 6, sl], rows_v[b, 7, sl]),
                        ),
                    )
                    out_v[b, rt, pl.ds(c * L, L)] = m
                return carry2

            lax.fori_loop(0, RT, rt_body, 0)
            pltpu.async_copy(
                out_v.at[b],
                out_hbm.at[pl.ds(t0 + o * RT, RT), d], osem)

            @pl.when(i + 2 < NGATH)
            def _():
                gather_start(i + 2, b)
        return carry

    lax.fori_loop(0, NGATH // 2, i2_body, 0)

    for b in range(2):
        pltpu.make_async_copy(
            out_v.at[b], out_hbm.at[pl.ds(t0, RT), 0], osem).wait()


@jax.jit
def _pooler(p_flat, idx_scaled):
    mesh = plsc.VectorSubcoreMesh(core_axis_name="c", subcore_axis_name="s")
    f = functools.partial(
        pl.kernel,
        out_type=jax.ShapeDtypeStruct((T, D, E), jnp.float32),
        mesh=mesh,
        scratch_types=[
            pltpu.VMEM((DG,), jnp.int32),
            pltpu.VMEM((OPW, DG), jnp.int32),
            pltpu.VMEM((2, G, RE), jnp.float32),
            pltpu.VMEM((2, RT, E), jnp.float32),
            pltpu.SemaphoreType.DMA,
            pltpu.SemaphoreType.DMA,
        ],
    )(_pooler_body)
    return f(p_flat, idx_scaled)


def kernel(P, subgroup_indices):
    p_flat = P.reshape(N * TQ, RE)
    idx_scaled = (subgroup_indices.astype(jnp.int32) * jnp.int32(TQ)).reshape(DG)
    out = _pooler(p_flat, idx_scaled)
    return out.reshape(1, T, D * E)


# speedup vs baseline: 1.4564x; 1.4564x over previous
import functools

import jax
import jax.numpy as jnp
from jax import lax
from jax.experimental import pallas as pl
from jax.experimental.pallas import tpu as pltpu
from jax.experimental.pallas import tpu_sc as plsc

N, T, E = 64, 512, 768
D, G = 8, 8
DG = D * G
NC, NS = 2, 16
NW = NC * NS
L = 16
EC = E // L

TTC = 384
TSC = T - TTC
TPW = TSC // NW
BT = 128


def _sc_body(p_hbm, idx_hbm, out_hbm, base_v, idx_all_v, rows_v, out_v,
             gsem, osem):
    cid = lax.axis_index("c")
    sid = lax.axis_index("s")
    wid = sid * NC + cid
    t0 = wid * TPW

    pltpu.sync_copy(idx_hbm, base_v)
    for tl in range(TPW):
        t = TTC + t0 + tl
        for k in range(DG // L):
            sl = pl.ds(k * L, L)
            idx_all_v[tl, sl] = base_v[sl] + t

    def gather_start(tl, b):
        pltpu.async_copy(p_hbm.at[idx_all_v.at[tl]], rows_v.at[b], gsem)

    def gather_wait(b):
        pltpu.make_async_copy(p_hbm.at[idx_all_v.at[0]], rows_v.at[b],
                              gsem).wait()

    gather_start(0, 0)
    gather_start(1, 1)

    def t2_body(tlo, carry):
        for b in range(2):
            tl = tlo * 2 + b
            gather_wait(b)
            @pl.when(tl >= 2)
            def _():
                pltpu.make_async_copy(out_v.at[b], out_hbm.at[t0], osem).wait()

            def d_body(d, carry2):
                r0 = d * G
                for c in range(EC):
                    sl = pl.ds(c * L, L)
                    m = jnp.maximum(
                        jnp.maximum(
                            jnp.maximum(rows_v[b, r0, sl], rows_v[b, r0 + 1, sl]),
                            jnp.maximum(rows_v[b, r0 + 2, sl], rows_v[b, r0 + 3, sl]),
                        ),
                        jnp.maximum(
                            jnp.maximum(rows_v[b, r0 + 4, sl], rows_v[b, r0 + 5, sl]),
                            jnp.maximum(rows_v[b, r0 + 6, sl], rows_v[b, r0 + 7, sl]),
                        ),
                    )
                    out_v[b, d, sl] = m
                return carry2

            lax.fori_loop(0, D, d_body, 0)
            pltpu.async_copy(out_v.at[b], out_hbm.at[t0 + tl], osem)

            @pl.when(tl + 2 < TPW)
            def _():
                gather_start(tl + 2, b)
        return carry

    lax.fori_loop(0, TPW // 2, t2_body, 0)

    for b in range(2):
        pltpu.make_async_copy(out_v.at[b], out_hbm.at[t0], osem).wait()


def _sc_pool(p_flat, idx_scaled):
    mesh = plsc.VectorSubcoreMesh(core_axis_name="c", subcore_axis_name="s")
    f = functools.partial(
        pl.kernel,
        out_type=jax.ShapeDtypeStruct((TSC, D, E), jnp.float32),
        mesh=mesh,
        scratch_types=[
            pltpu.VMEM((DG,), jnp.int32),
            pltpu.VMEM((TPW, DG), jnp.int32),
            pltpu.VMEM((2, DG, E), jnp.float32),
            pltpu.VMEM((2, D, E), jnp.float32),
            pltpu.SemaphoreType.DMA,
            pltpu.SemaphoreType.DMA,
        ],
    )(_sc_body)
    return f(p_flat, idx_scaled)


def _tc_body(idx_ref, p_ref, o_ref):
    g = pl.program_id(2)

    @pl.when(g == 0)
    def _():
        o_ref[...] = p_ref[0]

    @pl.when(g > 0)
    def _():
        o_ref[...] = jnp.maximum(o_ref[...], p_ref[0])


def _tc_pool(P, idx_flat):
    grid_spec = pltpu.PrefetchScalarGridSpec(
        num_scalar_prefetch=1,
        grid=(D, TTC // BT, G),
        in_specs=[
            pl.BlockSpec((1, BT, E), lambda d, t, g, idx: (idx[d * G + g], t, 0)),
        ],
        out_specs=pl.BlockSpec((BT, E), lambda d, t, g, idx: (t, d)),
    )
    return pl.pallas_call(
        _tc_body,
        grid_spec=grid_spec,
        out_shape=jax.ShapeDtypeStruct((TTC, D * E), jnp.float32),
    )(idx_flat, P)


@jax.jit
def _pooler(P, idx_flat):
    p_flat = P.reshape(N * T, E)
    idx_scaled = idx_flat * jnp.int32(T)
    out_tc = _tc_pool(P, idx_flat)
    out_sc = _sc_pool(p_flat, idx_scaled)
    out = jnp.concatenate([out_tc, out_sc.reshape(TSC, D * E)], axis=0)
    return out.reshape(1, T, D * E)


def kernel(P, subgroup_indices):
    idx_flat = subgroup_indices.astype(jnp.int32).reshape(DG)
    return _pooler(P, idx_flat)
